# baseline (speedup 1.0000x reference)
import jax
import jax.numpy as jnp
from jax.experimental import pallas as pl
from jax.experimental.pallas import tpu as pltpu

_DROP_WIDTH = 64
_STRIPES_NUM = 2

_NB = 8
_D = 4
_CHB = 4

_WNB = 8
_WD = 4


def _stripe_params(B, total_width):
    key = jax.random.key(42)
    k_dist, k_bgn = jax.random.split(key)
    distances = jax.random.randint(k_dist, (B, _STRIPES_NUM), 0, _DROP_WIDTH)
    u = jax.random.uniform(k_bgn, (B, _STRIPES_NUM))
    bgns = jnp.floor(u * (total_width - distances).astype(jnp.float32)).astype(
        jnp.int32
    )
    ends = bgns + distances.astype(jnp.int32)
    return jnp.concatenate([bgns, ends], axis=1)


def _copy_body(x_hbm, o_hbm, bufs, sin, sout):
    B = x_hbm.shape[0]
    nchk = B // _CHB

    def in_copy(i):
        k = i % _NB
        return pltpu.make_async_copy(
            x_hbm.at[pl.ds(i * _CHB, _CHB)], bufs.at[k], sin.at[k]
        )

    def out_copy(i):
        k = i % _NB
        return pltpu.make_async_copy(
            bufs.at[k], o_hbm.at[pl.ds(i * _CHB, _CHB)], sout.at[k]
        )

    for i in range(nchk + _D):
        if i < nchk:
            if i >= _NB:
                out_copy(i - _NB).wait()
            in_copy(i).start()
        j = i - _D
        if 0 <= j < nchk:
            in_copy(j).wait()
            out_copy(j).start()
    for j in range(nchk - _NB, nchk):
        out_copy(j).wait()


def _fixup_body(params_ref, x_hbm, big_hbm, o_hbm, wbufs, sin, sout):
    del big_hbm
    B, W, C = x_hbm.shape
    nwin = B * _STRIPES_NUM
    iota = jax.lax.broadcasted_iota(jnp.int32, (_DROP_WIDTH, 1), 0)

    def win_start(w):
        b, s = w // _STRIPES_NUM, w % _STRIPES_NUM
        return b, jnp.minimum(params_ref[b, s], W - _DROP_WIDTH)

    def in_copy(w):
        k = w % _WNB
        b, st = win_start(w)
        return pltpu.make_async_copy(
            x_hbm.at[b, pl.ds(st, _DROP_WIDTH), :], wbufs.at[k], sin.at[k]
        )

    def out_copy(w):
        k = w % _WNB
        b, st = win_start(w)
        return pltpu.make_async_copy(
            wbufs.at[k], o_hbm.at[b, pl.ds(st, _DROP_WIDTH), :], sout.at[k]
        )

    def apply_mask(w):
        k = w % _WNB
        b, st = win_start(w)
        idx = iota + st
        drop = (idx >= params_ref[b, 0]) & (idx < params_ref[b, _STRIPES_NUM])
        for t in range(1, _STRIPES_NUM):
            drop |= (idx >= params_ref[b, t]) & (
                idx < params_ref[b, _STRIPES_NUM + t]
            )
        keep = jnp.where(drop, 0.0, 1.0)
        wbufs[k] = wbufs[k] * keep

    for i in range(nwin + _WD):
        if i < nwin:
            if i >= _WNB:
                out_copy(i - _WNB).wait()
            in_copy(i).start()
        j = i - _WD
        if 0 <= j < nwin:
            in_copy(j).wait()
            apply_mask(j)
            out_copy(j).start()
    for j in range(nwin - _WNB, nwin):
        out_copy(j).wait()


@jax.jit
def kernel(input):
    B, W, C = input.shape
    params = _stripe_params(B, W)
    copied = pl.pallas_call(
        _copy_body,
        in_specs=[pl.BlockSpec(memory_space=pltpu.HBM)],
        out_specs=pl.BlockSpec(memory_space=pltpu.HBM),
        out_shape=jax.ShapeDtypeStruct((B, W, C), input.dtype),
        scratch_shapes=[
            pltpu.VMEM((_NB, _CHB, W, C), jnp.float32),
            pltpu.SemaphoreType.DMA((_NB,)),
            pltpu.SemaphoreType.DMA((_NB,)),
        ],
    )(input)
    return pl.pallas_call(
        _fixup_body,
        in_specs=[
            pl.BlockSpec(memory_space=pltpu.SMEM),
            pl.BlockSpec(memory_space=pltpu.HBM),
            pl.BlockSpec(memory_space=pltpu.HBM),
        ],
        out_specs=pl.BlockSpec(memory_space=pltpu.HBM),
        out_shape=jax.ShapeDtypeStruct((B, W, C), input.dtype),
        input_output_aliases={2: 0},
        scratch_shapes=[
            pltpu.VMEM((_WNB, _DROP_WIDTH, C), jnp.float32),
            pltpu.SemaphoreType.DMA((_WNB,)),
            pltpu.SemaphoreType.DMA((_WNB,)),
        ],
    )(params, input, copied)


# speedup vs baseline: 1.2314x; 1.2314x over previous
import jax
import jax.numpy as jnp
from jax.experimental import pallas as pl
from jax.experimental.pallas import tpu as pltpu

_DROP_WIDTH = 64
_STRIPES_NUM = 2

_NB = 8
_D = 4
_CHB = 4

_WNB = 32
_WD = 24


def _stripe_params(B, total_width):
    key = jax.random.key(42)
    k_dist, k_bgn = jax.random.split(key)
    distances = jax.random.randint(k_dist, (B, _STRIPES_NUM), 0, _DROP_WIDTH)
    u = jax.random.uniform(k_bgn, (B, _STRIPES_NUM))
    bgns = jnp.floor(u * (total_width - distances).astype(jnp.float32)).astype(
        jnp.int32
    )
    ends = bgns + distances.astype(jnp.int32)
    return jnp.concatenate([bgns, ends], axis=1)


def _copy_body(x_hbm, o_hbm, bufs, sin, sout):
    B = x_hbm.shape[0]
    nchk = B // _CHB

    def in_copy(i):
        k = i % _NB
        return pltpu.make_async_copy(
            x_hbm.at[pl.ds(i * _CHB, _CHB)], bufs.at[k], sin.at[k]
        )

    def out_copy(i):
        k = i % _NB
        return pltpu.make_async_copy(
            bufs.at[k], o_hbm.at[pl.ds(i * _CHB, _CHB)], sout.at[k]
        )

    for i in range(nchk + _D):
        if i < nchk:
            if i >= _NB:
                out_copy(i - _NB).wait()
            in_copy(i).start()
        j = i - _D
        if 0 <= j < nchk:
            in_copy(j).wait()
            out_copy(j).start()
    for j in range(nchk - _NB, nchk):
        out_copy(j).wait()


def _fixup_body(params_ref, x_hbm, big_hbm, o_hbm, wbufs, sin, sout):
    del big_hbm
    B, W, C = x_hbm.shape
    nwin = B * _STRIPES_NUM
    iota = jax.lax.broadcasted_iota(jnp.int32, (_DROP_WIDTH, 1), 0)

    def win_start(w):
        b, s = w // _STRIPES_NUM, w % _STRIPES_NUM
        return b, jnp.minimum(params_ref[b, s], W - _DROP_WIDTH)

    def in_copy(w):
        k = w % _WNB
        b, st = win_start(w)
        return pltpu.make_async_copy(
            x_hbm.at[b, pl.ds(st, _DROP_WIDTH), :], wbufs.at[k], sin.at[k]
        )

    def out_copy(w):
        k = w % _WNB
        b, st = win_start(w)
        return pltpu.make_async_copy(
            wbufs.at[k], o_hbm.at[b, pl.ds(st, _DROP_WIDTH), :], sout.at[k]
        )

    def apply_mask(w):
        k = w % _WNB
        b, st = win_start(w)
        idx = iota + st
        drop = (idx >= params_ref[b, 0]) & (idx < params_ref[b, _STRIPES_NUM])
        for t in range(1, _STRIPES_NUM):
            drop |= (idx >= params_ref[b, t]) & (
                idx < params_ref[b, _STRIPES_NUM + t]
            )
        keep = jnp.where(drop, 0.0, 1.0)
        wbufs[k] = wbufs[k] * keep

    for i in range(nwin + _WD):
        if i < nwin:
            if i >= _WNB:
                out_copy(i - _WNB).wait()
            in_copy(i).start()
        j = i - _WD
        if 0 <= j < nwin:
            in_copy(j).wait()
            apply_mask(j)
            out_copy(j).start()
    for j in range(nwin - _WNB, nwin):
        out_copy(j).wait()


@jax.jit
def kernel(input):
    B, W, C = input.shape
    params = _stripe_params(B, W)
    copied = pl.pallas_call(
        _copy_body,
        in_specs=[pl.BlockSpec(memory_space=pltpu.HBM)],
        out_specs=pl.BlockSpec(memory_space=pltpu.HBM),
        out_shape=jax.ShapeDtypeStruct((B, W, C), input.dtype),
        scratch_shapes=[
            pltpu.VMEM((_NB, _CHB, W, C), jnp.float32),
            pltpu.SemaphoreType.DMA((_NB,)),
            pltpu.SemaphoreType.DMA((_NB,)),
        ],
    )(input)
    return pl.pallas_call(
        _fixup_body,
        in_specs=[
            pl.BlockSpec(memory_space=pltpu.SMEM),
            pl.BlockSpec(memory_space=pltpu.HBM),
            pl.BlockSpec(memory_space=pltpu.HBM),
        ],
        out_specs=pl.BlockSpec(memory_space=pltpu.HBM),
        out_shape=jax.ShapeDtypeStruct((B, W, C), input.dtype),
        input_output_aliases={2: 0},
        scratch_shapes=[
            pltpu.VMEM((_WNB, _DROP_WIDTH, C), jnp.float32),
            pltpu.SemaphoreType.DMA((_WNB,)),
            pltpu.SemaphoreType.DMA((_WNB,)),
        ],
    )(params, input, copied)


# speedup vs baseline: 1.4493x; 1.1769x over previous
import jax
import jax.numpy as jnp
from jax.experimental import pallas as pl
from jax.experimental.pallas import tpu as pltpu

_DROP_WIDTH = 64
_STRIPES_NUM = 2

_NB = 8
_D = 4
_CHB = 4
_POWS = (32, 16, 8, 4, 2, 1)


def _stripe_params(B, total_width):
    key = jax.random.key(42)
    k_dist, k_bgn = jax.random.split(key)
    distances = jax.random.randint(k_dist, (B, _STRIPES_NUM), 0, _DROP_WIDTH)
    u = jax.random.uniform(k_bgn, (B, _STRIPES_NUM))
    bgns = jnp.floor(u * (total_width - distances).astype(jnp.float32)).astype(
        jnp.int32
    )
    ends = bgns + distances.astype(jnp.int32)
    return jnp.concatenate([bgns, ends], axis=1)


def _body(params_ref, x_hbm, o_hbm, bufs, zbuf, sin, sout, szero):
    B = x_hbm.shape[0]
    nchk = B // _CHB
    zbuf[...] = jnp.zeros(zbuf.shape, zbuf.dtype)

    def in_copy(i):
        k = i % _NB
        return pltpu.make_async_copy(
            x_hbm.at[pl.ds(i * _CHB, _CHB)], bufs.at[k], sin.at[k]
        )

    def out_copy(i):
        k = i % _NB
        return pltpu.make_async_copy(
            bufs.at[k], o_hbm.at[pl.ds(i * _CHB, _CHB)], sout.at[k]
        )

    def zero_walk(j, wait):
        for ii in range(_CHB):
            b = j * _CHB + ii
            for s in range(_STRIPES_NUM):
                bgn = params_ref[b, s]
                width = params_ref[b, _STRIPES_NUM + s] - bgn
                for p in _POWS:
                    off = bgn + (width & (63 ^ (2 * p - 1)))
                    d = pltpu.make_async_copy(
                        zbuf.at[pl.ds(0, p)],
                        o_hbm.at[b, pl.ds(off, p), :],
                        szero,
                    )

                    @pl.when((width & p) != 0)
                    def _():
                        if wait:
                            d.wait()
                        else:
                            d.start()

    for i in range(nchk + _D):
        if i < nchk:
            if i >= _NB:
                out_copy(i - _NB).wait()
                zero_walk(i - _NB, wait=False)
                if i - _NB >= 1:
                    zero_walk(i - _NB - 1, wait=True)
            in_copy(i).start()
        j = i - _D
        if 0 <= j < nchk:
            in_copy(j).wait()
            out_copy(j).start()
    for j in range(nchk - _NB, nchk):
        out_copy(j).wait()
        zero_walk(j, wait=False)
        zero_walk(j - 1, wait=True)
    zero_walk(nchk - 1, wait=True)


@jax.jit
def kernel(input):
    B, W, C = input.shape
    params = _stripe_params(B, W)
    return pl.pallas_call(
        _body,
        in_specs=[
            pl.BlockSpec(memory_space=pltpu.SMEM),
            pl.BlockSpec(memory_space=pltpu.HBM),
        ],
        out_specs=pl.BlockSpec(memory_space=pltpu.HBM),
        out_shape=jax.ShapeDtypeStruct((B, W, C), input.dtype),
        scratch_shapes=[
            pltpu.VMEM((_NB, _CHB, W, C), jnp.float32),
            pltpu.VMEM((_POWS[0], C), jnp.float32),
            pltpu.SemaphoreType.DMA((_NB,)),
            pltpu.SemaphoreType.DMA((_NB,)),
            pltpu.SemaphoreType.DMA,
        ],
    )(params, input)


# speedup vs baseline: 1.5144x; 1.0449x over previous
import jax
import jax.numpy as jnp
from jax.experimental import pallas as pl
from jax.experimental.pallas import tpu as pltpu

_DROP_WIDTH = 64
_STRIPES_NUM = 2
_BB = 8


def _stripe_params(B, total_width):
    key = jax.random.key(42)
    k_dist, k_bgn = jax.random.split(key)
    distances = jax.random.randint(k_dist, (B, _STRIPES_NUM), 0, _DROP_WIDTH)
    u = jax.random.uniform(k_bgn, (B, _STRIPES_NUM))
    bgns = jnp.floor(u * (total_width - distances).astype(jnp.float32)).astype(
        jnp.int32
    )
    ends = bgns + distances.astype(jnp.int32)
    return jnp.concatenate([bgns, ends], axis=1)


def _body(params_ref, x_ref, o_ref):
    bb = x_ref.shape[0]
    W = x_ref.shape[1]
    b_base = pl.program_id(0) * bb
    o_ref[...] = x_ref[...]
    iota = jax.lax.broadcasted_iota(jnp.int32, (_DROP_WIDTH, 1), 0)
    for i in range(bb):
        b = b_base + i
        for s in range(_STRIPES_NUM):
            st = jnp.minimum(params_ref[b, s], W - _DROP_WIDTH)
            idx = iota + st
            drop = (idx >= params_ref[b, 0]) & (idx < params_ref[b, _STRIPES_NUM])
            for t in range(1, _STRIPES_NUM):
                drop |= (idx >= params_ref[b, t]) & (
                    idx < params_ref[b, _STRIPES_NUM + t]
                )
            keep = jnp.where(drop, 0.0, 1.0)
            win = pl.ds(st, _DROP_WIDTH)
            o_ref[i, win, :] = x_ref[i, win, :] * keep


@jax.jit
def kernel(input):
    B, W, C = input.shape
    params = _stripe_params(B, W)
    return pl.pallas_call(
        _body,
        grid=(B // _BB,),
        in_specs=[
            pl.BlockSpec(memory_space=pltpu.SMEM),
            pl.BlockSpec((_BB, W, C), lambda b: (b, 0, 0)),
        ],
        out_specs=pl.BlockSpec((_BB, W, C), lambda b: (b, 0, 0)),
        out_shape=jax.ShapeDtypeStruct((B, W, C), input.dtype),
    )(params, input)
